# baseline (speedup 1.0000x reference)
import dataclasses
import functools
import jax
import jax.numpy as jnp
from jax import lax
from jax.experimental import pallas as pl
from jax.experimental.pallas import tpu as pltpu
from jax.experimental.pallas import tpu_sc as plsc

B, C, H, W, K = 8, 192, 128, 128, 19
HW = H * W
NEG_INF = float("-inf")
NEG = -3.0e38

NC, NS, L = 2, 16, 16
PT = HW // NS
NB_PER_CORE = B // NC
KP = 32
NCHUNK = PT // L
ROW = KP + L


def _sc_stats_body(preds_hbm, w_hbm, am_hbm, xch_hbm,
                   pbuf, s_loc, am_loc, e_loc, w_loc,
                   rowbuf, rscale, allbuf):
    cid = lax.axis_index("c")
    sid = lax.axis_index("s")
    pix0 = sid * PT
    row0 = sid * (PT // W)

    for bi in range(NB_PER_CORE):
        b = cid * NB_PER_CORE + bi
        pltpu.sync_copy(preds_hbm.at[b, :, pl.ds(row0, PT // W), :], pbuf)

        def c1(ci, mcar):
            r = ci // 8
            i = (ci % 8) * L
            m = jnp.full((L,), NEG, jnp.float32)
            amv = jnp.zeros((L,), jnp.int32)
            for k in range(K):
                v = pbuf[k, r, pl.ds(i, L)]
                gt = v > m
                amv = jnp.where(gt, jnp.full((L,), k, jnp.int32), amv)
                m = jnp.where(gt, v, m)
            s_loc[pl.ds(ci * L, L)] = m
            am_loc[pl.ds(ci * L, L)] = amv
            return jnp.maximum(mcar, m)

        mvec = lax.fori_loop(0, NCHUNK, c1, jnp.full((L,), NEG, jnp.float32))
        mt = jnp.full((L,), jnp.max(mvec, axis=0), jnp.float32)
        rowbuf[pl.ds(KP, L)] = mt

        def c2(ci, accs):
            sl = pl.ds(ci * L, L)
            ev = jnp.exp(s_loc[sl] - mt)
            e_loc[sl] = ev
            amv = am_loc[sl]
            return tuple(
                acc + jnp.where(amv == k, ev, 0.0)
                for k, acc in enumerate(accs))

        init = tuple(jnp.zeros((L,), jnp.float32) for _ in range(K))
        accs = lax.fori_loop(0, NCHUNK, c2, init)
        lane_ids = lax.iota(jnp.int32, L)
        v0 = jnp.zeros((L,), jnp.float32)
        v1 = jnp.zeros((L,), jnp.float32)
        for k in range(K):
            tot = jnp.sum(accs[k], axis=0)
            if k < L:
                v0 = jnp.where(lane_ids == k, tot, v0)
            else:
                v1 = jnp.where(lane_ids == (k - L), tot, v1)
        rowbuf[pl.ds(0, L)] = v0
        rowbuf[pl.ds(L, L)] = v1

        pltpu.sync_copy(rowbuf, xch_hbm.at[cid, sid])
        plsc.subcore_barrier()
        pltpu.sync_copy(xch_hbm.at[cid], allbuf)
        plsc.subcore_barrier()

        gmax = jnp.full((L,), NEG, jnp.float32)
        for t in range(NS):
            gmax = jnp.maximum(gmax, allbuf[t, pl.ds(KP, L)])
        gs0 = jnp.zeros((L,), jnp.float32)
        gs1 = jnp.zeros((L,), jnp.float32)
        for t in range(NS):
            sc_t = jnp.exp(allbuf[t, pl.ds(KP, L)] - gmax)
            gs0 = gs0 + allbuf[t, pl.ds(0, L)] * sc_t
            gs1 = gs1 + allbuf[t, pl.ds(L, L)] * sc_t
        myscale = jnp.exp(mt - gmax)
        rscale[pl.ds(0, L)] = myscale / gs0
        rscale[pl.ds(L, L)] = myscale / gs1

        def c3(ci, carry):
            sl = pl.ds(ci * L, L)
            rv = plsc.load_gather(rscale, [am_loc[sl]])
            w_loc[sl] = e_loc[sl] * rv
            return carry

        lax.fori_loop(0, NCHUNK, c3, jnp.int32(0))

        pltpu.sync_copy(w_loc, w_hbm.at[b, 0, pl.ds(pix0, PT)])
        pltpu.sync_copy(am_loc, am_hbm.at[b, 0, pl.ds(pix0, PT)])


def _sc_stats(preds):
    mesh = plsc.VectorSubcoreMesh(core_axis_name="c", subcore_axis_name="s")
    cp = pltpu.CompilerParams()
    if "needs_layout_passes" in pltpu.CompilerParams.__dataclass_fields__:
        cp = dataclasses.replace(cp, needs_layout_passes=False)
    kern = pl.kernel(
        _sc_stats_body,
        out_type=[
            jax.ShapeDtypeStruct((B, 1, HW), jnp.float32),
            jax.ShapeDtypeStruct((B, 1, HW), jnp.int32),
            jax.ShapeDtypeStruct((NC, NS, ROW), jnp.float32),
        ],
        mesh=mesh,
        scratch_types=[
            pltpu.VMEM((K, PT // W, W), jnp.float32),
            pltpu.VMEM((PT,), jnp.float32),
            pltpu.VMEM((PT,), jnp.int32),
            pltpu.VMEM((PT,), jnp.float32),
            pltpu.VMEM((PT,), jnp.float32),
            pltpu.VMEM((ROW,), jnp.float32),
            pltpu.VMEM((KP,), jnp.float32),
            pltpu.VMEM((NS, ROW), jnp.float32),
        ],
        compiler_params=cp,
    )
    w, am, _xch = kern(preds)
    return w, am


def _phase_a_body(w_ref, am_ref, x_ref, cent_ref):
    wv = w_ref[0, 0]
    am = am_ref[0, 0]
    xb = x_ref[0].astype(jnp.bfloat16).reshape(C, HW)
    kio = jax.lax.broadcasted_iota(jnp.int32, (K, HW), 0)
    mt = jnp.where(am[None, :] == kio, wv[None, :], 0.0).astype(jnp.bfloat16)
    cent = jax.lax.dot_general(
        xb, mt, (((1,), (1,)), ((), ())),
        preferred_element_type=jnp.float32)
    cent_ref[0] = cent


HB = 32
PB = HB * W


def _phase_b_body(cent_ref, am_ref, out_ref):
    cent = cent_ref[0]
    am = am_ref[0, 0]
    kio = jax.lax.broadcasted_iota(jnp.int32, (K, PB), 0)
    onehot = (am[None, :] == kio).astype(jnp.bfloat16)
    res = jax.lax.dot_general(
        cent.astype(jnp.bfloat16), onehot, (((1,), (0,)), ((), ())),
        preferred_element_type=jnp.float32)
    out_ref[0] = res.reshape(C, HB, W)


def kernel(x, preds):
    w, am = _sc_stats(preds)

    cent = pl.pallas_call(
        _phase_a_body,
        grid=(B,),
        in_specs=[
            pl.BlockSpec((1, 1, HW), lambda b: (b, 0, 0)),
            pl.BlockSpec((1, 1, HW), lambda b: (b, 0, 0)),
            pl.BlockSpec((1, C, H, W), lambda b: (b, 0, 0, 0)),
        ],
        out_specs=pl.BlockSpec((1, C, K), lambda b: (b, 0, 0)),
        out_shape=jax.ShapeDtypeStruct((B, C, K), jnp.float32),
    )(w, am, x)

    out = pl.pallas_call(
        _phase_b_body,
        grid=(B, H // HB),
        in_specs=[
            pl.BlockSpec((1, C, K), lambda b, j: (b, 0, 0)),
            pl.BlockSpec((1, 1, PB), lambda b, j: (b, 0, j)),
        ],
        out_specs=pl.BlockSpec((1, C, HB, W), lambda b, j: (b, 0, j, 0)),
        out_shape=jax.ShapeDtypeStruct((B, C, H, W), jnp.float32),
    )(cent, am)

    return out


# speedup vs baseline: 1.0409x; 1.0409x over previous
import jax
import jax.numpy as jnp
from jax.experimental import pallas as pl
from jax.experimental import pallas as _pl_unused
from jax.experimental.pallas import tpu as pltpu

B, C, H, W, K = 8, 192, 128, 128, 19
HW = H * W
NEG_INF = float("-inf")
HH = H // 2
PB = HH * W


def _fused_body(preds_ref, x_ref, out_ref, cent_s, am_s):
    j = pl.program_id(1)

    @pl.when(j == 0)
    def _phase_a():
        p = preds_ref[0].reshape(K, HW)
        s = jnp.max(p, axis=0)
        am = jnp.argmax(p, axis=0).astype(jnp.int32)
        kio = jax.lax.broadcasted_iota(jnp.int32, (K, HW), 0)
        mask = am[None, :] == kio
        m = jnp.max(jnp.where(mask, s[None, :], NEG_INF), axis=1)
        m_pp = jnp.sum(jnp.where(mask, m[:, None], 0.0), axis=0)
        e = jnp.exp(s - m_pp)
        d = jnp.sum(jnp.where(mask, e[None, :], 0.0), axis=1)
        d_pp = jnp.sum(jnp.where(mask, d[:, None], 0.0), axis=0)
        wgt = e / d_pp
        mt = jnp.where(mask, wgt[None, :], 0.0).astype(jnp.bfloat16)
        xb = x_ref[0].astype(jnp.bfloat16).reshape(C, HW)
        cent = jax.lax.dot_general(
            xb, mt, (((1,), (1,)), ((), ())),
            preferred_element_type=jnp.float32)
        cent_s[...] = cent.astype(jnp.bfloat16)
        am_s[...] = am.reshape(1, HW)

    half = j * PB
    am_h = am_s[0, pl.ds(half, PB)]
    kio2 = jax.lax.broadcasted_iota(jnp.int32, (K, PB), 0)
    onehot = (am_h[None, :] == kio2).astype(jnp.bfloat16)
    res = jax.lax.dot_general(
        cent_s[...], onehot, (((1,), (0,)), ((), ())),
        preferred_element_type=jnp.float32)
    out_ref[0] = res.reshape(C, HH, W)


def kernel(x, preds):
    out = pl.pallas_call(
        _fused_body,
        grid=(B, 2),
        in_specs=[
            pl.BlockSpec((1, K, H, W), lambda b, j: (b, 0, 0, 0)),
            pl.BlockSpec((1, C, H, W), lambda b, j: (b, 0, 0, 0)),
        ],
        out_specs=pl.BlockSpec((1, C, HH, W), lambda b, j: (b, 0, j, 0)),
        out_shape=jax.ShapeDtypeStruct((B, C, H, W), jnp.float32),
        scratch_shapes=[
            pltpu.VMEM((C, K), jnp.bfloat16),
            pltpu.VMEM((1, HW), jnp.int32),
        ],
    )(preds, x)
    return out


# speedup vs baseline: 1.4571x; 1.3998x over previous
import jax
import jax.numpy as jnp
from jax.experimental import pallas as pl

B, C, H, W, K = 8, 192, 128, 128, 19
HW = H * W
NEG_INF = float("-inf")


def _phase_a_body(preds_ref, x_ref, cent_ref, am_ref):
    p = preds_ref[0].reshape(K, HW)
    s = jnp.max(p, axis=0)
    am = jnp.argmax(p, axis=0).astype(jnp.int32)
    smax = jnp.max(s)
    e = jnp.exp(s - smax)
    kio = jax.lax.broadcasted_iota(jnp.int32, (K, HW), 0)
    mask = am[None, :] == kio
    d = jnp.sum(jnp.where(mask, e[None, :], 0.0), axis=1)
    mt = jnp.where(mask, e[None, :], 0.0).astype(jnp.bfloat16)
    xb = x_ref[0].astype(jnp.bfloat16).reshape(C, HW)
    cent = jax.lax.dot_general(
        xb, mt, (((1,), (1,)), ((), ())),
        preferred_element_type=jnp.float32)
    cent_ref[0] = cent / jnp.maximum(d, 1e-30)[None, :]
    am_ref[0, 0] = am


HB = 32
PB = HB * W


def _phase_b_body(cent_ref, am_ref, out_ref):
    cent = cent_ref[0]
    am = am_ref[0, 0]
    kio = jax.lax.broadcasted_iota(jnp.int32, (K, PB), 0)
    onehot = (am[None, :] == kio).astype(jnp.bfloat16)
    res = jax.lax.dot_general(
        cent.astype(jnp.bfloat16), onehot, (((1,), (0,)), ((), ())),
        preferred_element_type=jnp.float32)
    out_ref[0] = res.reshape(C, HB, W)


def kernel(x, preds):
    cent, am = pl.pallas_call(
        _phase_a_body,
        grid=(B,),
        in_specs=[
            pl.BlockSpec((1, K, H, W), lambda b: (b, 0, 0, 0)),
            pl.BlockSpec((1, C, H, W), lambda b: (b, 0, 0, 0)),
        ],
        out_specs=[
            pl.BlockSpec((1, C, K), lambda b: (b, 0, 0)),
            pl.BlockSpec((1, 1, HW), lambda b: (b, 0, 0)),
        ],
        out_shape=[
            jax.ShapeDtypeStruct((B, C, K), jnp.float32),
            jax.ShapeDtypeStruct((B, 1, HW), jnp.int32),
        ],
    )(preds, x)

    out = pl.pallas_call(
        _phase_b_body,
        grid=(B, H // HB),
        in_specs=[
            pl.BlockSpec((1, C, K), lambda b, j: (b, 0, 0)),
            pl.BlockSpec((1, 1, PB), lambda b, j: (b, 0, j)),
        ],
        out_specs=pl.BlockSpec((1, C, HB, W), lambda b, j: (b, 0, j, 0)),
        out_shape=jax.ShapeDtypeStruct((B, C, H, W), jnp.float32),
    )(cent, am)

    return out
